# baseline (speedup 1.0000x reference)
import functools
import math

import jax
import jax.numpy as jnp
from jax import lax
from jax.experimental import pallas as pl
from jax.experimental.pallas import tpu as pltpu
from jax.experimental.pallas import tpu_sc as plsc

_L = 16
_NW = 32
_NROW = 162
_STRIDE = _NROW + 1
_KPAD = 176
_HWORDS = _L * _STRIDE
_HALLOC = ((_HWORDS + 127) // 128) * 128
_CBIT = 22
_BIAS = 1 << _CBIT
_FIX = 256.0
_SU = 8


def _sc_histogram(samples, nodes, chunk):
    mesh = plsc.VectorSubcoreMesh(core_axis_name="c", subcore_axis_name="s")
    n = samples.shape[0]
    n_nodes = nodes.shape[0]
    tail_s = n - (_NW - 1) * chunk
    tail_n = n_nodes - (_NW - 1) * chunk
    assert 0 < tail_s <= chunk and 0 < tail_n <= chunk

    @functools.partial(
        pl.kernel,
        mesh=mesh,
        out_type=jax.ShapeDtypeStruct((_NW, 2 * _KPAD), jnp.float32),
        scratch_types=[
            pltpu.VMEM((2 * chunk,), jnp.float32),
            pltpu.VMEM((_HALLOC,), jnp.int32),
            pltpu.VMEM((2 * _KPAD,), jnp.float32),
            pltpu.SemaphoreType.DMA,
            pltpu.SemaphoreType.DMA,
        ],
        compiler_params=pltpu.CompilerParams(needs_layout_passes=False),
    )
    def hist_kernel(s_hbm, t_hbm, out_hbm, x_v, hist_v, red_v, sem_s, sem_t):
        wid = lax.axis_index("s") * 2 + lax.axis_index("c")
        base = wid * chunk
        is_last = wid == _NW - 1

        zeros = jnp.zeros((_L,), jnp.int32)
        zu = 8

        def zero_hist():
            def zbody(i, carry):
                for k in range(zu):
                    off = (i * zu + k) * _L
                    hist_v[pl.ds(off, _L)] = zeros
                return carry
            lax.fori_loop(0, _HALLOC // (_L * zu), zbody, 0)

        lane = lax.iota(jnp.int32, _L)
        base_s = lane * _STRIDE - (1 << (_CBIT - 5))
        base_n = base_s + 1
        fma_c = jnp.float32(0.5 + _BIAS)
        scale = jnp.float32(_FIX)
        n_vec = chunk // _L

        def make_body(x_off, lane_base):
            def body(i, carry):
                for k in range(_SU):
                    v = i * _SU + k
                    x = x_v[pl.ds(x_off + v * _L, _L)]
                    q2 = (x * scale + fma_c).astype(jnp.int32)
                    fi = lax.shift_right_logical(q2, 5) + lane_base
                    plsc.addupdate_scatter(hist_v, [fi], q2)
                return carry
            return body

        def make_body_masked(x_off, lane_base, limit):
            def body(i, carry):
                for k in range(_SU):
                    v = i * _SU + k
                    x = x_v[pl.ds(x_off + v * _L, _L)]
                    mask = (lane + v * _L) < limit
                    q2 = (x * scale + fma_c).astype(jnp.int32)
                    fi = lax.shift_right_logical(q2, 5) + lane_base
                    plsc.addupdate_scatter(hist_v, [fi], q2, mask=mask)
                return carry
            return body

        @pl.when(jnp.logical_not(is_last))
        def _():
            c1 = pltpu.async_copy(s_hbm.at[pl.ds(base, chunk)],
                                  x_v.at[pl.ds(0, chunk)], sem_s)
            c2 = pltpu.async_copy(t_hbm.at[pl.ds(base, chunk)],
                                  x_v.at[pl.ds(chunk, chunk)], sem_t)
            zero_hist()
            c1.wait()
            lax.fori_loop(0, n_vec // _SU, make_body(0, base_s), 0)
            c2.wait()
            lax.fori_loop(0, n_vec // _SU, make_body(chunk, base_n), 0)

        @pl.when(is_last)
        def _():
            c1 = pltpu.async_copy(s_hbm.at[pl.ds((_NW - 1) * chunk, tail_s)],
                                  x_v.at[pl.ds(0, tail_s)], sem_s)
            c2 = pltpu.async_copy(t_hbm.at[pl.ds((_NW - 1) * chunk, tail_n)],
                                  x_v.at[pl.ds(chunk, tail_n)], sem_t)
            zero_hist()
            c1.wait()
            lax.fori_loop(0, n_vec // _SU,
                          make_body_masked(0, base_s, jnp.int32(tail_s)), 0)
            c2.wait()
            lax.fori_loop(0, n_vec // _SU,
                          make_body_masked(chunk, base_n, jnp.int32(tail_n)), 0)

        fix_mask = jnp.int32((1 << _CBIT) - 1)
        inv_fix = jnp.float32(1.0 / _FIX)
        n_red = _KPAD // _L
        valid_last = (lane + (n_red - 1) * _L) < _NROW

        def rbody(c, carry):
            t0 = hist_v[pl.ds(c * _L, _L)]
            acc_c = lax.shift_right_logical(t0, _CBIT)
            acc_s = jnp.bitwise_and(t0, fix_mask)
            for l in range(1, _L):
                t = hist_v[pl.ds(l * _STRIDE + c * _L, _L)]
                acc_c = acc_c + lax.shift_right_logical(t, _CBIT)
                acc_s = acc_s + jnp.bitwise_and(t, fix_mask)
            fc = acc_c.astype(jnp.float32)
            fs = acc_s.astype(jnp.float32) * inv_fix
            is_tail = c == n_red - 1
            fc = jnp.where(jnp.logical_and(is_tail,
                                           jnp.logical_not(valid_last)),
                           jnp.float32(0.0), fc)
            fs = jnp.where(jnp.logical_and(is_tail,
                                           jnp.logical_not(valid_last)),
                           jnp.float32(0.0), fs)
            red_v[pl.ds(c * _L, _L)] = fc
            red_v[pl.ds(_KPAD + c * _L, _L)] = fs
            return carry

        lax.fori_loop(0, n_red, rbody, 0)

        pltpu.sync_copy(red_v, out_hbm.at[wid])

    return hist_kernel(samples, nodes)


def _tc_finish(n, red, theta_mu, theta_sigma, eps):
    m_total = float(2 * n - 1)
    nm1 = float(n - 1)
    half_log_2pi = 0.5 * math.log(2.0 * math.pi)
    node_lo = 81

    def body(red_ref, mu_ref, ts_ref, eps_ref, out_ref):
        total = jnp.sum(red_ref[...], axis=0, keepdims=True)
        c = total[:, :_KPAD]
        s = total[:, _KPAD:]
        ii = lax.broadcasted_iota(jnp.int32, (_KPAD, _KPAD), 0)
        jj = lax.broadcasted_iota(jnp.int32, (_KPAD, _KPAD), 1)
        tri = (ii < jj).astype(jnp.float32)
        gbase = jax.lax.dot_general(
            c, tri, (((1,), (0,)), ((), ())),
            preferred_element_type=jnp.float32)
        sum_t = jnp.sum(s[:, node_lo:])
        sum1 = -jnp.sum((gbase + (c - 1.0) * 0.5) * s) + m_total * sum_t

        mu = mu_ref[...]
        ts = ts_ref[...]
        ep = eps_ref[...]
        z = mu + jnp.exp(ts) * ep
        inv_theta = jnp.exp(-z)
        elbo = (-sum1 * inv_theta - nm1 * z + z + ts
                + half_log_2pi + 0.5 * ep * ep)
        out_ref[...] = elbo

    return pl.pallas_call(
        body,
        out_shape=jax.ShapeDtypeStruct((1, 1), jnp.float32),
    )(red, theta_mu, theta_sigma, eps)


def kernel(node_heights, sampling_times, theta_mu, theta_sigma, eps):
    n = sampling_times.shape[0]
    grain = _L * _SU
    chunk = ((n + _NW * grain - 1) // (_NW * grain)) * grain
    red = _sc_histogram(sampling_times, node_heights, chunk)
    return _tc_finish(n, red, theta_mu, theta_sigma, eps)


# speedup vs baseline: 1.0124x; 1.0124x over previous
import functools
import math

import jax
import jax.numpy as jnp
from jax import lax
from jax.experimental import pallas as pl
from jax.experimental.pallas import tpu as pltpu
from jax.experimental.pallas import tpu_sc as plsc

_L = 16
_NW = 32
_NROW = 162
_STRIDE = _NROW + 1
_KPAD = 176
_HWORDS = _L * _STRIDE
_HALLOC = ((_HWORDS + 127) // 128) * 128
_CBIT = 22
_BIAS = 1 << _CBIT
_FIX = 256.0
_SU = 8


def _sc_histogram(samples, nodes, chunk):
    mesh = plsc.VectorSubcoreMesh(core_axis_name="c", subcore_axis_name="s")
    n = samples.shape[0]
    n_nodes = nodes.shape[0]
    tail_s = n - (_NW - 1) * chunk
    tail_n = n_nodes - (_NW - 1) * chunk
    assert 0 < tail_s <= chunk and 0 < tail_n <= chunk

    @functools.partial(
        pl.kernel,
        mesh=mesh,
        out_type=jax.ShapeDtypeStruct((_NW, 2 * _KPAD), jnp.float32),
        scratch_types=[
            pltpu.VMEM((2 * chunk,), jnp.float32),
            pltpu.VMEM((_HALLOC,), jnp.int32),
            pltpu.VMEM((2 * _KPAD,), jnp.float32),
            pltpu.SemaphoreType.DMA,
            pltpu.SemaphoreType.DMA,
        ],
        compiler_params=pltpu.CompilerParams(needs_layout_passes=False),
    )
    def hist_kernel(s_hbm, t_hbm, out_hbm, x_v, hist_v, red_v, sem_s, sem_t):
        wid = lax.axis_index("s") * 2 + lax.axis_index("c")
        base = wid * chunk
        is_last = wid == _NW - 1

        zeros = jnp.zeros((_L,), jnp.int32)
        zu = 8

        def zero_hist():
            def zbody(i, carry):
                for k in range(zu):
                    off = (i * zu + k) * _L
                    hist_v[pl.ds(off, _L)] = zeros
                return carry
            lax.fori_loop(0, _HALLOC // (_L * zu), zbody, 0)

        lane = lax.iota(jnp.int32, _L)
        base_s = lane * _STRIDE - (1 << (_CBIT - 5))
        base_n = base_s + 1
        fma_c = jnp.float32(0.5 + _BIAS)
        scale = jnp.float32(_FIX)
        n_vec = chunk // _L

        def make_body(x_off, lane_base):
            def body(i, carry):
                for k in range(_SU):
                    v = i * _SU + k
                    x = x_v[pl.ds(x_off + v * _L, _L)]
                    q2 = (x * scale + fma_c).astype(jnp.int32)
                    fi = lax.shift_right_logical(q2, 5) + lane_base
                    plsc.addupdate_scatter(hist_v, [fi], q2)
                return carry
            return body

        def make_body_masked(x_off, lane_base, limit):
            def body(i, carry):
                for k in range(_SU):
                    v = i * _SU + k
                    x = x_v[pl.ds(x_off + v * _L, _L)]
                    mask = (lane + v * _L) < limit
                    q2 = (x * scale + fma_c).astype(jnp.int32)
                    fi = lax.shift_right_logical(q2, 5) + lane_base
                    plsc.addupdate_scatter(hist_v, [fi], q2, mask=mask)
                return carry
            return body

        @pl.when(jnp.logical_not(is_last))
        def _():
            c1 = pltpu.async_copy(s_hbm.at[pl.ds(base, chunk)],
                                  x_v.at[pl.ds(0, chunk)], sem_s)
            c2 = pltpu.async_copy(t_hbm.at[pl.ds(base, chunk)],
                                  x_v.at[pl.ds(chunk, chunk)], sem_t)
            zero_hist()
            c1.wait()
            lax.fori_loop(0, n_vec // _SU, make_body(0, base_s), 0)
            c2.wait()
            lax.fori_loop(0, n_vec // _SU, make_body(chunk, base_n), 0)

        @pl.when(is_last)
        def _():
            c1 = pltpu.async_copy(s_hbm.at[pl.ds((_NW - 1) * chunk, tail_s)],
                                  x_v.at[pl.ds(0, tail_s)], sem_s)
            c2 = pltpu.async_copy(t_hbm.at[pl.ds((_NW - 1) * chunk, tail_n)],
                                  x_v.at[pl.ds(chunk, tail_n)], sem_t)
            zero_hist()
            c1.wait()
            lax.fori_loop(0, n_vec // _SU,
                          make_body_masked(0, base_s, jnp.int32(tail_s)), 0)
            c2.wait()
            lax.fori_loop(0, n_vec // _SU,
                          make_body_masked(chunk, base_n, jnp.int32(tail_n)), 0)

        fix_mask = jnp.int32((1 << _CBIT) - 1)
        inv_fix = jnp.float32(1.0 / _FIX)
        n_red = _KPAD // _L
        valid_last = (lane + (n_red - 1) * _L) < _NROW

        def rbody(c, carry):
            t = [hist_v[pl.ds(l * _STRIDE + c * _L, _L)] for l in range(_L)]
            p = [t[2 * j] + t[2 * j + 1] for j in range(_L // 2)]
            cs = [lax.shift_right_logical(w, _CBIT) for w in p]
            ss = [jnp.bitwise_and(w, fix_mask) for w in p]
            while len(cs) > 1:
                cs = [cs[2 * j] + cs[2 * j + 1] for j in range(len(cs) // 2)]
                ss = [ss[2 * j] + ss[2 * j + 1] for j in range(len(ss) // 2)]
            fc = cs[0].astype(jnp.float32)
            fs = ss[0].astype(jnp.float32) * inv_fix
            is_tail = c == n_red - 1
            fc = jnp.where(jnp.logical_and(is_tail,
                                           jnp.logical_not(valid_last)),
                           jnp.float32(0.0), fc)
            fs = jnp.where(jnp.logical_and(is_tail,
                                           jnp.logical_not(valid_last)),
                           jnp.float32(0.0), fs)
            red_v[pl.ds(c * _L, _L)] = fc
            red_v[pl.ds(_KPAD + c * _L, _L)] = fs
            return carry

        lax.fori_loop(0, n_red, rbody, 0)

        pltpu.sync_copy(red_v, out_hbm.at[wid])

    return hist_kernel(samples, nodes)


def _tc_finish(n, red, theta_mu, theta_sigma, eps):
    m_total = float(2 * n - 1)
    nm1 = float(n - 1)
    half_log_2pi = 0.5 * math.log(2.0 * math.pi)
    node_lo = 81

    def body(red_ref, mu_ref, ts_ref, eps_ref, out_ref):
        total = jnp.sum(red_ref[...], axis=0, keepdims=True)
        c = total[:, :_KPAD]
        s = total[:, _KPAD:]
        ii = lax.broadcasted_iota(jnp.int32, (_KPAD, _KPAD), 0)
        jj = lax.broadcasted_iota(jnp.int32, (_KPAD, _KPAD), 1)
        tri = (ii < jj).astype(jnp.float32)
        gbase = jax.lax.dot_general(
            c, tri, (((1,), (0,)), ((), ())),
            preferred_element_type=jnp.float32)
        sum_t = jnp.sum(s[:, node_lo:])
        sum1 = -jnp.sum((gbase + (c - 1.0) * 0.5) * s) + m_total * sum_t

        mu = mu_ref[...]
        ts = ts_ref[...]
        ep = eps_ref[...]
        z = mu + jnp.exp(ts) * ep
        inv_theta = jnp.exp(-z)
        elbo = (-sum1 * inv_theta - nm1 * z + z + ts
                + half_log_2pi + 0.5 * ep * ep)
        out_ref[...] = elbo

    return pl.pallas_call(
        body,
        out_shape=jax.ShapeDtypeStruct((1, 1), jnp.float32),
    )(red, theta_mu, theta_sigma, eps)


def kernel(node_heights, sampling_times, theta_mu, theta_sigma, eps):
    n = sampling_times.shape[0]
    grain = _L * _SU
    chunk = ((n + _NW * grain - 1) // (_NW * grain)) * grain
    red = _sc_histogram(sampling_times, node_heights, chunk)
    return _tc_finish(n, red, theta_mu, theta_sigma, eps)
